# Initial kernel scaffold; baseline (speedup 1.0000x reference)
import jax
import jax.numpy as jnp
from jax.experimental import pallas as pl


def kernel(x, edge_index, W_gcn, b_gcn, W_ih, W_hh, b_ih, b_hh, h0):
    raise NotImplementedError("write your pallas kernel here")


# speedup vs baseline: 18.9256x; 18.9256x over previous
import functools

import jax
import jax.numpy as jnp
from jax import lax
from jax.experimental import pallas as pl
from jax.experimental.pallas import tpu as pltpu
from jax.experimental.pallas import tpu_sc as plsc

N = 10000
E = 320000
DI = 128
DO = 128
H3 = 384
NW = 32
CHUNK = 128
CPW = 79
EPW = CHUNK * CPW
E_PAD = NW * EPW
CNT_PAD = 10240
ACC_PAD = 10112

_mesh = plsc.VectorSubcoreMesh(core_axis_name="c", subcore_axis_name="s")


@functools.partial(
    pl.kernel,
    out_type=jax.ShapeDtypeStruct((2 * CNT_PAD,), jnp.float32),
    mesh=_mesh,
    scratch_types=[
        pltpu.VMEM((CPW, CHUNK), jnp.int32),
        pltpu.VMEM((CHUNK,), jnp.float32),
        pltpu.VMEM((640,), jnp.float32),
        pltpu.VMEM_SHARED((CNT_PAD,), jnp.float32),
    ],
)
def _deg_kernel(dst_hbm, ones_hbm, zeros_hbm, out_hbm, idx_v, ones_v, zv, cnt_s):
    c = lax.axis_index("c")
    s = lax.axis_index("s")
    wid = s * 2 + c
    pltpu.sync_copy(zeros_hbm, zv)
    pltpu.sync_copy(zv, cnt_s.at[pl.ds(s * 640, 640)])
    pltpu.sync_copy(ones_hbm, ones_v)
    pltpu.sync_copy(dst_hbm.at[wid], idx_v)
    plsc.subcore_barrier()

    def body(j, carry):
        pltpu.sync_copy(ones_v, cnt_s.at[idx_v.at[j]], add=True)
        return carry

    lax.fori_loop(0, CPW, body, 0)
    plsc.subcore_barrier()
    pltpu.sync_copy(cnt_s.at[pl.ds(s * 640, 640)], zv)
    pltpu.sync_copy(zv, out_hbm.at[pl.ds(c * CNT_PAD + s * 640, 640)])


@functools.partial(
    pl.kernel,
    out_type=jax.ShapeDtypeStruct((2, ACC_PAD, DO), jnp.float32),
    mesh=_mesh,
    scratch_types=[
        pltpu.VMEM((CPW, CHUNK), jnp.int32),
        pltpu.VMEM((CPW, CHUNK), jnp.int32),
        pltpu.VMEM((CHUNK, DO), jnp.float32),
        pltpu.VMEM_SHARED((ACC_PAD, DO), jnp.float32),
    ],
)
def _gcn_gather(src_hbm, dst_hbm, y_hbm, zeros_hbm, out_hbm,
                src_v, dst_v, rows_v, acc_s):
    c = lax.axis_index("c")
    s = lax.axis_index("s")
    wid = s * 2 + c
    pltpu.sync_copy(zeros_hbm, rows_v)
    for k in range(4):
        pltpu.sync_copy(rows_v, acc_s.at[pl.ds(s * 632 + k * 128, 128)])
    pltpu.sync_copy(rows_v.at[pl.ds(0, 120)],
                    acc_s.at[pl.ds(s * 632 + 512, 120)])
    pltpu.sync_copy(src_hbm.at[wid], src_v)
    pltpu.sync_copy(dst_hbm.at[wid], dst_v)
    plsc.subcore_barrier()

    def body(j, carry):
        pltpu.sync_copy(y_hbm.at[src_v.at[j]], rows_v)
        pltpu.sync_copy(rows_v, acc_s.at[dst_v.at[j]], add=True)
        return carry

    lax.fori_loop(0, CPW, body, 0)
    plsc.subcore_barrier()
    for k in range(4):
        pltpu.sync_copy(acc_s.at[pl.ds(s * 632 + k * 128, 128)], rows_v)
        pltpu.sync_copy(rows_v, out_hbm.at[c, pl.ds(s * 632 + k * 128, 128)])
    pltpu.sync_copy(acc_s.at[pl.ds(s * 632 + 512, 120)],
                    rows_v.at[pl.ds(0, 120)])
    pltpu.sync_copy(rows_v.at[pl.ds(0, 120)],
                    out_hbm.at[c, pl.ds(s * 632 + 512, 120)])


BLK = 1000
GRID = N // BLK


def _tc1_body(x_ref, wg_ref, c0_ref, c1_ref, y_ref, dinv_ref):
    xw = jnp.dot(x_ref[:], wg_ref[:], preferred_element_type=jnp.float32)
    deg = c0_ref[:] + c1_ref[:] + 1.0
    dinv = lax.rsqrt(deg)
    y_ref[:] = xw * dinv
    dinv_ref[:] = dinv


_tc1 = pl.pallas_call(
    _tc1_body,
    grid=(GRID,),
    in_specs=[
        pl.BlockSpec((BLK, DI), lambda i: (i, 0)),
        pl.BlockSpec((DI, DO), lambda i: (0, 0)),
        pl.BlockSpec((BLK, 1), lambda i: (i, 0)),
        pl.BlockSpec((BLK, 1), lambda i: (i, 0)),
    ],
    out_specs=[
        pl.BlockSpec((BLK, DO), lambda i: (i, 0)),
        pl.BlockSpec((BLK, 1), lambda i: (i, 0)),
    ],
    out_shape=[
        jax.ShapeDtypeStruct((N, DO), jnp.float32),
        jax.ShapeDtypeStruct((N, 1), jnp.float32),
    ],
)


def _tc2_body(a0_ref, a1_ref, y_ref, dv_ref, bg_ref, wih_ref, bih_ref,
              whh_ref, bhh_ref, h0_ref, outs_ref, hf_ref, gi_ref, h_ref):
    i = pl.program_id(0)

    @pl.when(i == 0)
    def _init():
        h_ref[:] = h0_ref[:]

    gc = (a0_ref[:] + a1_ref[:] + y_ref[:]) * dv_ref[:] + bg_ref[:]
    gi_ref[:] = (jnp.dot(gc, wih_ref[:], preferred_element_type=jnp.float32)
                 + bih_ref[:])
    whh = whh_ref[:]
    bhh = bhh_ref[:]

    def step(t, h):
        gh = jnp.dot(h, whh, preferred_element_type=jnp.float32) + bhh
        git = gi_ref[pl.ds(t, 1), :]
        r = jax.nn.sigmoid(git[:, :DO] + gh[:, :DO])
        z = jax.nn.sigmoid(git[:, DO:2 * DO] + gh[:, DO:2 * DO])
        n_ = jnp.tanh(git[:, 2 * DO:] + r * gh[:, 2 * DO:])
        h = (1.0 - z) * n_ + z * h
        outs_ref[pl.ds(t, 1), :] = h
        return h

    h = lax.fori_loop(0, BLK, step, h_ref[:])
    h_ref[:] = h
    hf_ref[:] = h


_tc2 = pl.pallas_call(
    _tc2_body,
    grid=(GRID,),
    in_specs=[
        pl.BlockSpec((BLK, DO), lambda i: (i, 0)),
        pl.BlockSpec((BLK, DO), lambda i: (i, 0)),
        pl.BlockSpec((BLK, DO), lambda i: (i, 0)),
        pl.BlockSpec((BLK, 1), lambda i: (i, 0)),
        pl.BlockSpec((1, DO), lambda i: (0, 0)),
        pl.BlockSpec((DO, H3), lambda i: (0, 0)),
        pl.BlockSpec((1, H3), lambda i: (0, 0)),
        pl.BlockSpec((DO, H3), lambda i: (0, 0)),
        pl.BlockSpec((1, H3), lambda i: (0, 0)),
        pl.BlockSpec((1, DO), lambda i: (0, 0)),
    ],
    out_specs=[
        pl.BlockSpec((BLK, DO), lambda i: (i, 0)),
        pl.BlockSpec((1, DO), lambda i: (0, 0)),
    ],
    out_shape=[
        jax.ShapeDtypeStruct((N, DO), jnp.float32),
        jax.ShapeDtypeStruct((1, DO), jnp.float32),
    ],
    scratch_shapes=[
        pltpu.VMEM((BLK, H3), jnp.float32),
        pltpu.VMEM((1, DO), jnp.float32),
    ],
)


def kernel(x, edge_index, W_gcn, b_gcn, W_ih, W_hh, b_ih, b_hh, h0):
    src = edge_index[0]
    dst = edge_index[1]
    pad = E_PAD - E
    src_p = jnp.concatenate([src, jnp.zeros((pad,), jnp.int32)])
    dst_p = jnp.concatenate([dst, jnp.full((pad,), N, jnp.int32)])
    src_p = src_p.reshape(NW, CPW, CHUNK)
    dst_p = dst_p.reshape(NW, CPW, CHUNK)

    ones = jnp.ones((CHUNK,), jnp.float32)
    z1 = jnp.zeros((640,), jnp.float32)
    cnt = _deg_kernel(dst_p, ones, z1).reshape(2, CNT_PAD)
    c0 = cnt[0, :N, None]
    c1 = cnt[1, :N, None]

    y, dinv = _tc1(x, W_gcn, c0, c1)

    z2 = jnp.zeros((CHUNK, DO), jnp.float32)
    acc = _gcn_gather(src_p, dst_p, y, z2)

    outs, hf = _tc2(acc[0, :N], acc[1, :N], y, dinv,
                    b_gcn.reshape(1, DO), W_ih.T, b_ih.reshape(1, H3),
                    W_hh.T, b_hh.reshape(1, H3), h0[0])
    return outs[None], hf[None]
